# baseline (speedup 1.0000x reference)
import functools

import jax
import jax.numpy as jnp
from jax import lax
from jax.experimental import pallas as pl
from jax.experimental.pallas import tpu as pltpu
from jax.experimental.pallas import tpu_sc as plsc

INPUT_DIM = 768
HIDDEN = INPUT_DIM // 2
NUM_EXPERTS = 64
HALF = NUM_EXPERTS // 2
N_TOKENS = 32768
BT = 4096

NC = 2
NS = 16
NW = NC * NS
TPW = N_TOKENS // NW
L = 16


def _top2(vals, lane_f):
    m1 = jnp.max(vals, axis=-1, keepdims=True)
    i1f = jnp.min(jnp.where(vals == m1, lane_f, float(NUM_EXPERTS)),
                  axis=-1, keepdims=True)
    masked = jnp.where(lane_f == i1f, -jnp.inf, vals)
    m2 = jnp.max(masked, axis=-1, keepdims=True)
    i2f = jnp.min(jnp.where(masked == m2, lane_f, float(NUM_EXPERTS)),
                  axis=-1, keepdims=True)
    return m1, i1f, m2, i2f


def _cand_kernel(x_ref, w1_ref, b1_ref, w2_ref, b2_ref, cand_ref):
    h = jnp.dot(x_ref[:], w1_ref[:], preferred_element_type=jnp.float32)
    h = jnp.maximum(h + b1_ref[:], 0.0)
    logits = jnp.dot(h, w2_ref[:], preferred_element_type=jnp.float32)
    logits = logits + b2_ref[:]

    lane_f = jax.lax.broadcasted_iota(jnp.int32, logits.shape, 1).astype(
        jnp.float32)
    in_a = lane_f < float(HALF)
    ma1, ia1, ma2, ia2 = _top2(jnp.where(in_a, logits, -jnp.inf), lane_f)
    mb1, ib1, mb2, ib2 = _top2(jnp.where(in_a, -jnp.inf, logits), lane_f)
    cand = jnp.concatenate(
        [ma1, ia1, ma2, ia2, mb1, ib1, mb2, ib2], axis=1)
    cand_ref[:] = cand.T


def _tc_candidates(x, W1, b1, W2, b2):
    n = x.shape[0]
    return pl.pallas_call(
        _cand_kernel,
        grid=(n // BT,),
        in_specs=[
            pl.BlockSpec((BT, INPUT_DIM), lambda i: (i, 0)),
            pl.BlockSpec((INPUT_DIM, HIDDEN), lambda i: (0, 0)),
            pl.BlockSpec((1, HIDDEN), lambda i: (0, 0)),
            pl.BlockSpec((HIDDEN, NUM_EXPERTS), lambda i: (0, 0)),
            pl.BlockSpec((1, NUM_EXPERTS), lambda i: (0, 0)),
        ],
        out_specs=pl.BlockSpec((8, BT), lambda i: (0, i)),
        out_shape=jax.ShapeDtypeStruct((8, n), jnp.float32),
        compiler_params=pltpu.CompilerParams(
            dimension_semantics=("parallel",),
        ),
    )(x, W1, b1.reshape(1, HIDDEN), W2, b2.reshape(1, NUM_EXPERTS))


def _sc_body(cand_hbm, ow_hbm, oi_hbm, vm, ow, oi):
    wid = lax.axis_index("s") * NC + lax.axis_index("c")
    base = wid * TPW
    pltpu.sync_copy(cand_hbm.at[:, pl.ds(base, TPW)], vm)

    def merge(t, _):
        tt = t * L
        ma1 = vm[0, pl.ds(tt, L)]
        ia1 = vm[1, pl.ds(tt, L)]
        ma2 = vm[2, pl.ds(tt, L)]
        ia2 = vm[3, pl.ds(tt, L)]
        mb1 = vm[4, pl.ds(tt, L)]
        ib1 = vm[5, pl.ds(tt, L)]
        mb2 = vm[6, pl.ds(tt, L)]
        ib2 = vm[7, pl.ds(tt, L)]
        awins = ma1 >= mb1
        m1 = jnp.where(awins, ma1, mb1)
        i1 = jnp.where(awins, ia1, ib1)
        ca = jnp.where(awins, ma2, ma1)
        cia = jnp.where(awins, ia2, ia1)
        cb = jnp.where(awins, mb1, mb2)
        cib = jnp.where(awins, ib1, ib2)
        a2wins = ca >= cb
        m2 = jnp.where(a2wins, ca, cb)
        i2 = jnp.where(a2wins, cia, cib)
        e2 = jnp.exp(m2 - m1)
        inv = 1.0 / (1.0 + e2)
        ow[0, pl.ds(tt, L)] = inv
        ow[1, pl.ds(tt, L)] = e2 * inv
        oi[0, pl.ds(tt, L)] = i1.astype(jnp.int32)
        oi[1, pl.ds(tt, L)] = i2.astype(jnp.int32)
        return 0

    lax.fori_loop(0, TPW // L, merge, 0)
    pltpu.sync_copy(ow, ow_hbm.at[:, pl.ds(base, TPW)])
    pltpu.sync_copy(oi, oi_hbm.at[:, pl.ds(base, TPW)])


@functools.cache
def _sc_merge():
    return pl.kernel(
        _sc_body,
        out_type=[
            jax.ShapeDtypeStruct((2, N_TOKENS), jnp.float32),
            jax.ShapeDtypeStruct((2, N_TOKENS), jnp.int32),
        ],
        mesh=plsc.VectorSubcoreMesh(
            core_axis_name="c", subcore_axis_name="s",
            num_cores=NC, num_subcores=NS),
        scratch_types=[
            pltpu.VMEM((8, TPW), jnp.float32),
            pltpu.VMEM((2, TPW), jnp.float32),
            pltpu.VMEM((2, TPW), jnp.int32),
        ],
    )


@jax.jit
def kernel(x, W1, b1, W2, b2):
    cand = _tc_candidates(x, W1, b1, W2, b2)
    ow, oi = _sc_merge()(cand)
    return (ow.T, oi.T)


# speedup vs baseline: 1.0704x; 1.0704x over previous
import functools

import jax
import jax.numpy as jnp
from jax import lax
from jax.experimental import pallas as pl
from jax.experimental.pallas import tpu as pltpu
from jax.experimental.pallas import tpu_sc as plsc

INPUT_DIM = 768
HIDDEN = INPUT_DIM // 2
NUM_EXPERTS = 64
N_TOKENS = 32768
BT = 4096
NCHUNK = 4
CHUNK = N_TOKENS // NCHUNK

NC = 2
NS = 16
NW = NC * NS
TPW = CHUNK // NW
L = 16


def _logits_kernel(x_ref, w1_ref, b1_ref, w2_ref, b2_ref, lt_ref):
    h = jnp.dot(x_ref[:], w1_ref[:], preferred_element_type=jnp.float32)
    h = jnp.maximum(h + b1_ref[:], 0.0)
    logits = jnp.dot(h, w2_ref[:], preferred_element_type=jnp.float32)
    logits = logits + b2_ref[:]
    lt_ref[:] = logits.T


def _tc_logits_chunk(c, x, W1, b1, W2, b2):
    blocks_per_chunk = CHUNK // BT
    return pl.pallas_call(
        _logits_kernel,
        grid=(blocks_per_chunk,),
        in_specs=[
            pl.BlockSpec((BT, INPUT_DIM),
                         lambda i, c=c: (c * (CHUNK // BT) + i, 0)),
            pl.BlockSpec((INPUT_DIM, HIDDEN), lambda i: (0, 0)),
            pl.BlockSpec((1, HIDDEN), lambda i: (0, 0)),
            pl.BlockSpec((HIDDEN, NUM_EXPERTS), lambda i: (0, 0)),
            pl.BlockSpec((1, NUM_EXPERTS), lambda i: (0, 0)),
        ],
        out_specs=pl.BlockSpec((NUM_EXPERTS, BT), lambda i: (0, i)),
        out_shape=jax.ShapeDtypeStruct((NUM_EXPERTS, CHUNK), jnp.float32),
        compiler_params=pltpu.CompilerParams(
            dimension_semantics=("parallel",),
        ),
    )(x, W1, b1.reshape(1, HIDDEN), W2, b2.reshape(1, NUM_EXPERTS))


def _sc_body(lt_hbm, ow_hbm, oi_hbm, vm, ow, oi):
    wid = lax.axis_index("s") * NC + lax.axis_index("c")
    base = wid * TPW
    pltpu.sync_copy(lt_hbm.at[:, pl.ds(base, TPW)], vm)

    def outer(t, _):
        tt = t * L

        def inner(e, carry):
            m1, i1, m2, i2 = carry
            v = vm[e, pl.ds(tt, L)]
            ev = jnp.full((L,), e, jnp.int32)
            gt1 = v > m1
            gt2 = v > m2
            m2n = jnp.where(gt1, m1, jnp.where(gt2, v, m2))
            i2n = jnp.where(gt1, i1, jnp.where(gt2, ev, i2))
            m1n = jnp.maximum(v, m1)
            i1n = jnp.where(gt1, ev, i1)
            return m1n, i1n, m2n, i2n

        neg = jnp.full((L,), -jnp.inf, jnp.float32)
        zero = jnp.zeros((L,), jnp.int32)
        m1, i1, m2, i2 = lax.fori_loop(0, NUM_EXPERTS, inner,
                                       (neg, zero, neg, zero))
        e2 = jnp.exp(m2 - m1)
        inv = 1.0 / (1.0 + e2)
        ow[0, pl.ds(tt, L)] = inv
        ow[1, pl.ds(tt, L)] = e2 * inv
        oi[0, pl.ds(tt, L)] = i1
        oi[1, pl.ds(tt, L)] = i2
        return 0

    lax.fori_loop(0, TPW // L, outer, 0)
    pltpu.sync_copy(ow, ow_hbm.at[:, pl.ds(base, TPW)])
    pltpu.sync_copy(oi, oi_hbm.at[:, pl.ds(base, TPW)])


@functools.cache
def _sc_topk():
    return pl.kernel(
        _sc_body,
        out_type=[
            jax.ShapeDtypeStruct((2, CHUNK), jnp.float32),
            jax.ShapeDtypeStruct((2, CHUNK), jnp.int32),
        ],
        mesh=plsc.VectorSubcoreMesh(
            core_axis_name="c", subcore_axis_name="s",
            num_cores=NC, num_subcores=NS),
        scratch_types=[
            pltpu.VMEM((NUM_EXPERTS, TPW), jnp.float32),
            pltpu.VMEM((2, TPW), jnp.float32),
            pltpu.VMEM((2, TPW), jnp.int32),
        ],
    )


@jax.jit
def kernel(x, W1, b1, W2, b2):
    sc = _sc_topk()
    ows, ois = [], []
    for c in range(NCHUNK):
        lt = _tc_logits_chunk(c, x, W1, b1, W2, b2)
        ow, oi = sc(lt)
        ows.append(ow)
        ois.append(oi)
    gate_w = jnp.concatenate(ows, axis=1).T
    gate_i = jnp.concatenate(ois, axis=1).T
    return (gate_w, gate_i)


# speedup vs baseline: 1.4427x; 1.3478x over previous
import jax
import jax.numpy as jnp
from jax.experimental import pallas as pl
from jax.experimental.pallas import tpu as pltpu

INPUT_DIM = 768
HIDDEN = INPUT_DIM // 2
NUM_EXPERTS = 64
BT = 4096


def _gate_kernel(x_ref, w1_ref, b1_ref, w2_ref, b2_ref, out_ref):
    h = jnp.dot(x_ref[:], w1_ref[:], preferred_element_type=jnp.float32)
    h = jnp.maximum(h + b1_ref[:], 0.0)
    logits = jnp.dot(h, w2_ref[:], preferred_element_type=jnp.float32)
    logits = logits + b2_ref[:]

    lane_f = jax.lax.broadcasted_iota(jnp.int32, logits.shape, 1).astype(
        jnp.float32)
    m1 = jnp.max(logits, axis=-1, keepdims=True)
    i1f = jnp.min(jnp.where(logits == m1, lane_f, float(NUM_EXPERTS)),
                  axis=-1, keepdims=True)
    masked = jnp.where(lane_f == i1f, -jnp.inf, logits)
    m2 = jnp.max(masked, axis=-1, keepdims=True)
    i2f = jnp.min(jnp.where(masked == m2, lane_f, float(NUM_EXPERTS)),
                  axis=-1, keepdims=True)

    e = jnp.exp(m2 - m1)
    inv = 1.0 / (1.0 + e)
    out = jnp.concatenate([inv, e * inv, i1f, i2f], axis=1)
    out_ref[:] = out.T


@jax.jit
def kernel(x, W1, b1, W2, b2):
    n = x.shape[0]
    out = pl.pallas_call(
        _gate_kernel,
        grid=(n // BT,),
        in_specs=[
            pl.BlockSpec((BT, INPUT_DIM), lambda i: (i, 0)),
            pl.BlockSpec((INPUT_DIM, HIDDEN), lambda i: (0, 0)),
            pl.BlockSpec((1, HIDDEN), lambda i: (0, 0)),
            pl.BlockSpec((HIDDEN, NUM_EXPERTS), lambda i: (0, 0)),
            pl.BlockSpec((1, NUM_EXPERTS), lambda i: (0, 0)),
        ],
        out_specs=pl.BlockSpec((4, BT), lambda i: (0, i)),
        out_shape=jax.ShapeDtypeStruct((4, n), jnp.float32),
        compiler_params=pltpu.CompilerParams(
            dimension_semantics=("parallel",),
        ),
    )(x, W1, b1.reshape(1, HIDDEN), W2, b2.reshape(1, NUM_EXPERTS))
    return (out[:2].T, out[2:].T.astype(jnp.int32))


# speedup vs baseline: 1.4645x; 1.0151x over previous
import jax
import jax.numpy as jnp
from jax.experimental import pallas as pl
from jax.experimental.pallas import tpu as pltpu

INPUT_DIM = 768
HIDDEN = INPUT_DIM // 2
NUM_EXPERTS = 64
BT = 2048


def _gate_kernel(x_ref, w1_ref, b1_ref, w2_ref, b2_ref, out_ref):
    h = jnp.dot(x_ref[:], w1_ref[:], preferred_element_type=jnp.float32)
    h = jnp.maximum(h + b1_ref[:], 0.0)
    logits = jnp.dot(h, w2_ref[:], preferred_element_type=jnp.float32)
    logits = logits + b2_ref[:]

    lane_f = jax.lax.broadcasted_iota(jnp.int32, logits.shape, 1).astype(
        jnp.float32)
    m1 = jnp.max(logits, axis=-1, keepdims=True)
    i1f = jnp.min(jnp.where(logits == m1, lane_f, float(NUM_EXPERTS)),
                  axis=-1, keepdims=True)
    masked = jnp.where(lane_f == i1f, -jnp.inf, logits)
    m2 = jnp.max(masked, axis=-1, keepdims=True)
    i2f = jnp.min(jnp.where(masked == m2, lane_f, float(NUM_EXPERTS)),
                  axis=-1, keepdims=True)

    e = jnp.exp(m2 - m1)
    inv = 1.0 / (1.0 + e)
    out = jnp.concatenate([inv, e * inv, i1f, i2f], axis=1)
    out_ref[:] = out.T


@jax.jit
def kernel(x, W1, b1, W2, b2):
    n = x.shape[0]
    out = pl.pallas_call(
        _gate_kernel,
        grid=(n // BT,),
        in_specs=[
            pl.BlockSpec((BT, INPUT_DIM), lambda i: (i, 0)),
            pl.BlockSpec((INPUT_DIM, HIDDEN), lambda i: (0, 0)),
            pl.BlockSpec((1, HIDDEN), lambda i: (0, 0)),
            pl.BlockSpec((HIDDEN, NUM_EXPERTS), lambda i: (0, 0)),
            pl.BlockSpec((1, NUM_EXPERTS), lambda i: (0, 0)),
        ],
        out_specs=pl.BlockSpec((4, BT), lambda i: (0, i)),
        out_shape=jax.ShapeDtypeStruct((4, n), jnp.float32),
        compiler_params=pltpu.CompilerParams(
            dimension_semantics=("parallel",),
        ),
    )(x, W1, b1.reshape(1, HIDDEN), W2, b2.reshape(1, NUM_EXPERTS))
    return (out[:2].T, out[2:].T.astype(jnp.int32))
